# baseline (speedup 1.0000x reference)
import functools

import jax
import jax.numpy as jnp
from jax import lax
from jax.experimental import pallas as pl
from jax.experimental.pallas import tpu as pltpu
from jax.experimental.pallas import tpu_sc as plsc

_NUM_USERS = 100000
_VOCAB = 1000000
_NC, _NS = 2, 16
_NW = _NC * _NS
_CHUNK = 128
_L = 16
_FIRE = 8


def _body(lab3_hbm, uids_hbm, embf_hbm, out_hbm,
          uids_v, idx1_v, ids_v, int_v, idx2_v, emb_v, sem, bpw, dim):
    wid = lax.axis_index("s") * _NC + lax.axis_index("c")
    base = wid * bpw
    n3 = 3 * bpw
    nel = n3 * dim

    pltpu.sync_copy(uids_hbm.at[pl.ds(base, bpw)], uids_v)

    for j in range(3):
        for k in range(bpw // _L):
            p = j * bpw + k * _L
            u = uids_v[pl.ds(k * _L, _L)]
            idx1_v[p // _CHUNK, pl.ds(p % _CHUNK, _L)] = u + j * _NUM_USERS

    cps = [pltpu.async_copy(lab3_hbm.at[idx1_v.at[i]], ids_v.at[i], sem)
           for i in range(n3 // _CHUNK)]
    for cp in cps:
        cp.wait()

    three = jnp.full((_L,), 3, jnp.int32)
    iota = lax.iota(jnp.int32, _L)
    for k in range(n3 // _L):
        p = k * _L
        pos = iota + p
        b = lax.div(pos, three)
        j = lax.rem(pos, three)
        q = j * bpw + b
        qr = lax.shift_right_logical(q, 7)
        qc = lax.bitwise_and(q, _CHUNK - 1)
        int_v[pl.ds(p, _L)] = plsc.load_gather(ids_v, [qr, qc])

    ramp = iota * _VOCAB

    def expand(k, _):
        e = plsc.load_gather(int_v, [jnp.full((_L,), k, jnp.int32)])
        idx2_v[lax.div(k, 8), pl.ds(lax.rem(k, 8) * _L, _L)] = ramp + e
        return _

    lax.fori_loop(0, n3, expand, 0)

    nch = nel // _CHUNK
    for g in range(0, nch, _FIRE):
        cps = [pltpu.async_copy(embf_hbm.at[idx2_v.at[i]],
                                emb_v.at[pl.ds(i * _CHUNK, _CHUNK)], sem)
               for i in range(g, min(g + _FIRE, nch))]
        for cp in cps:
            cp.wait()

    pltpu.sync_copy(emb_v, out_hbm.at[pl.ds(wid * nel, nel)])


@jax.jit
def kernel(uids, train_labels, item_embeddings):
    batch = uids.shape[0]
    dim = item_embeddings.shape[1]
    hist = train_labels.shape[1]
    bpw = batch // _NW
    lab3 = train_labels.T[hist - 3:hist].reshape(-1)
    embf = item_embeddings.T.reshape(-1)

    run = pl.kernel(
        functools.partial(_body, bpw=bpw, dim=dim),
        out_type=jax.ShapeDtypeStruct((3 * batch * dim,), jnp.float32),
        mesh=plsc.VectorSubcoreMesh(core_axis_name="c", subcore_axis_name="s"),
        compiler_params=pltpu.CompilerParams(
            needs_layout_passes=False, use_tc_tiling_on_sc=False),
        scratch_types=[
            pltpu.VMEM((bpw,), jnp.int32),
            pltpu.VMEM((3 * bpw // _CHUNK, _CHUNK), jnp.int32),
            pltpu.VMEM((3 * bpw // _CHUNK, _CHUNK), jnp.int32),
            pltpu.VMEM((3 * bpw,), jnp.int32),
            pltpu.VMEM((3 * bpw * dim // _CHUNK, _CHUNK), jnp.int32),
            pltpu.VMEM((3 * bpw * dim,), jnp.float32),
            pltpu.SemaphoreType.DMA,
        ],
    )
    out = run(lab3, uids, embf)
    return out.reshape(batch, 3 * dim)


# speedup vs baseline: 4.2434x; 4.2434x over previous
import functools

import jax
import jax.numpy as jnp
from jax import lax
from jax.experimental import pallas as pl
from jax.experimental.pallas import tpu as pltpu
from jax.experimental.pallas import tpu_sc as plsc

_NUM_USERS = 100000
_VOCAB = 1000000
_NC, _NS = 2, 16
_NW = _NC * _NS
_CHUNK = 128
_L = 16
_FIRE = 8


def _body(lab3_hbm, uids_hbm, embf_hbm, out_hbm,
          uids_v, idx1_v, ids_v, int_v, idx2_v, emb_v, sem, bpw, dim):
    wid = lax.axis_index("s") * _NC + lax.axis_index("c")
    base = wid * bpw
    n3 = 3 * bpw
    nel = n3 * dim

    pltpu.sync_copy(uids_hbm.at[pl.ds(base, bpw)], uids_v)

    for j in range(3):
        for k in range(bpw // _L):
            p = j * bpw + k * _L
            u = uids_v[pl.ds(k * _L, _L)]
            idx1_v[p // _CHUNK, pl.ds(p % _CHUNK, _L)] = u + j * _NUM_USERS

    cps = [pltpu.async_copy(lab3_hbm.at[idx1_v.at[i]], ids_v.at[i], sem)
           for i in range(n3 // _CHUNK)]
    for cp in cps:
        cp.wait()

    three = jnp.full((_L,), 3, jnp.int32)
    iota = lax.iota(jnp.int32, _L)
    for k in range(n3 // _L):
        p = k * _L
        pos = iota + p
        b = lax.div(pos, three)
        j = lax.rem(pos, three)
        q = j * bpw + b
        qr = lax.shift_right_logical(q, 7)
        qc = lax.bitwise_and(q, _CHUNK - 1)
        int_v[pl.ds(p, _L)] = plsc.load_gather(ids_v, [qr, qc])

    wvec = jnp.full((_L,), _WCH, jnp.int32)
    dconst = (lax.shift_right_logical(iota, 3) * (8 * _WCH)
              + lax.bitwise_and(iota, 7) * _WCH)

    def expand(k, _):
        e = plsc.load_gather(int_v, [jnp.full((_L,), k, jnp.int32)])
        ch = lax.div(e, wvec)
        r = lax.rem(e, wvec)
        idx2_v[lax.div(k, 8), pl.ds(lax.rem(k, 8) * _L, _L)] = (
            ch * (16 * _WCH) + r + dconst)
        return _

    lax.fori_loop(0, n3, expand, 0)

    nch = nel // _CHUNK
    for g in range(0, nch, _FIRE):
        cps = [pltpu.async_copy(embf_hbm.at[idx2_v.at[i]],
                                emb_v.at[pl.ds(i * _CHUNK, _CHUNK)], sem)
               for i in range(g, min(g + _FIRE, nch))]
        for cp in cps:
            cp.wait()

    pltpu.sync_copy(emb_v, out_hbm.at[pl.ds(wid * nel, nel)])


_WCH = 7808


def _retile_body(in_ref, out_ref):
    out_ref[...] = in_ref[...].reshape(8 * _WCH // 128, 128)


def _retile_tc(emb_t):
    dim, vocab = emb_t.shape
    nch = (vocab + _WCH - 1) // _WCH
    rows = 8 * _WCH // 128
    return pl.pallas_call(
        _retile_body,
        grid=(dim // 8, nch),
        in_specs=[pl.BlockSpec((8, _WCH), lambda g, c: (g, c))],
        out_specs=pl.BlockSpec((rows, 128), lambda g, c: (c * (dim // 8) + g, 0)),
        out_shape=jax.ShapeDtypeStruct(((dim // 8) * nch * rows, 128),
                                       jnp.float32),
    )(emb_t)


@jax.jit
def kernel(uids, train_labels, item_embeddings):
    batch = uids.shape[0]
    dim = item_embeddings.shape[1]
    hist = train_labels.shape[1]
    bpw = batch // _NW
    lab3 = train_labels.T[hist - 3:hist].reshape(-1)
    embf = _retile_tc(item_embeddings.T).reshape(-1)

    run = pl.kernel(
        functools.partial(_body, bpw=bpw, dim=dim),
        out_type=jax.ShapeDtypeStruct((3 * batch * dim,), jnp.float32),
        mesh=plsc.VectorSubcoreMesh(core_axis_name="c", subcore_axis_name="s"),
        compiler_params=pltpu.CompilerParams(
            needs_layout_passes=False, use_tc_tiling_on_sc=False),
        scratch_types=[
            pltpu.VMEM((bpw,), jnp.int32),
            pltpu.VMEM((3 * bpw // _CHUNK, _CHUNK), jnp.int32),
            pltpu.VMEM((3 * bpw // _CHUNK, _CHUNK), jnp.int32),
            pltpu.VMEM((3 * bpw,), jnp.int32),
            pltpu.VMEM((3 * bpw * dim // _CHUNK, _CHUNK), jnp.int32),
            pltpu.VMEM((3 * bpw * dim,), jnp.float32),
            pltpu.SemaphoreType.DMA,
        ],
    )
    out = run(lab3, uids, embf)
    return out.reshape(batch, 3 * dim)


# speedup vs baseline: 4.2470x; 1.0009x over previous
import functools

import jax
import jax.numpy as jnp
from jax import lax
from jax.experimental import pallas as pl
from jax.experimental.pallas import tpu as pltpu
from jax.experimental.pallas import tpu_sc as plsc

_NUM_USERS = 100000
_VOCAB = 1000000
_NC, _NS = 2, 16
_NW = _NC * _NS
_CHUNK = 128
_L = 16
_FIRE = 8


def _body(lab3_hbm, uids_hbm, embf_hbm, out_hbm,
          uids_v, idx1_v, ids_v, int_v, idx2_v, emb_v, sem, bpw, dim):
    wid = lax.axis_index("s") * _NC + lax.axis_index("c")
    base = wid * bpw
    n3 = 3 * bpw
    nel = n3 * dim

    pltpu.sync_copy(uids_hbm.at[pl.ds(base, bpw)], uids_v)

    for j in range(3):
        for k in range(bpw // _L):
            p = j * bpw + k * _L
            u = uids_v[pl.ds(k * _L, _L)]
            idx1_v[p // _CHUNK, pl.ds(p % _CHUNK, _L)] = u + j * _NUM_USERS

    cps = [pltpu.async_copy(lab3_hbm.at[idx1_v.at[i]], ids_v.at[i], sem)
           for i in range(n3 // _CHUNK)]
    for cp in cps:
        cp.wait()

    three = jnp.full((_L,), 3, jnp.int32)
    iota = lax.iota(jnp.int32, _L)
    for k in range(n3 // _L):
        p = k * _L
        pos = iota + p
        b = lax.div(pos, three)
        j = lax.rem(pos, three)
        q = j * bpw + b
        qr = lax.shift_right_logical(q, 7)
        qc = lax.bitwise_and(q, _CHUNK - 1)
        int_v[pl.ds(p, _L)] = plsc.load_gather(ids_v, [qr, qc])

    wvec = jnp.full((_L,), _WCH, jnp.int32)
    dconst = (lax.shift_right_logical(iota, 3) * (8 * _WCH)
              + lax.bitwise_and(iota, 7) * 128)

    def expand(k, _):
        e = plsc.load_gather(int_v, [jnp.full((_L,), k, jnp.int32)])
        ch = lax.div(e, wvec)
        r = lax.rem(e, wvec)
        idx2_v[lax.div(k, 8), pl.ds(lax.rem(k, 8) * _L, _L)] = (
            ch * (16 * _WCH)
            + lax.shift_left(lax.shift_right_logical(r, 7), 10)
            + lax.bitwise_and(r, 127)
            + dconst)
        return _

    lax.fori_loop(0, n3, expand, 0)

    nch = nel // _CHUNK
    for g in range(0, nch, _FIRE):
        cps = [pltpu.async_copy(embf_hbm.at[idx2_v.at[i]],
                                emb_v.at[pl.ds(i * _CHUNK, _CHUNK)], sem)
               for i in range(g, min(g + _FIRE, nch))]
        for cp in cps:
            cp.wait()

    pltpu.sync_copy(emb_v, out_hbm.at[pl.ds(wid * nel, nel)])


_WCH = 7808


def _retile_body(in_ref, out_ref):
    out_ref[...] = (in_ref[...]
                    .reshape(8, _WCH // 128, 128)
                    .swapaxes(0, 1)
                    .reshape(8 * _WCH // 128, 128))


def _retile_tc(emb_t):
    dim, vocab = emb_t.shape
    nch = (vocab + _WCH - 1) // _WCH
    rows = 8 * _WCH // 128
    return pl.pallas_call(
        _retile_body,
        grid=(dim // 8, nch),
        in_specs=[pl.BlockSpec((8, _WCH), lambda g, c: (g, c))],
        out_specs=pl.BlockSpec((rows, 128), lambda g, c: (c * (dim // 8) + g, 0)),
        out_shape=jax.ShapeDtypeStruct(((dim // 8) * nch * rows, 128),
                                       jnp.float32),
    )(emb_t)


@jax.jit
def kernel(uids, train_labels, item_embeddings):
    batch = uids.shape[0]
    dim = item_embeddings.shape[1]
    hist = train_labels.shape[1]
    bpw = batch // _NW
    lab3 = train_labels.T[hist - 3:hist].reshape(-1)
    embf = _retile_tc(item_embeddings.T).reshape(-1)

    run = pl.kernel(
        functools.partial(_body, bpw=bpw, dim=dim),
        out_type=jax.ShapeDtypeStruct((3 * batch * dim,), jnp.float32),
        mesh=plsc.VectorSubcoreMesh(core_axis_name="c", subcore_axis_name="s"),
        compiler_params=pltpu.CompilerParams(
            needs_layout_passes=False, use_tc_tiling_on_sc=False),
        scratch_types=[
            pltpu.VMEM((bpw,), jnp.int32),
            pltpu.VMEM((3 * bpw // _CHUNK, _CHUNK), jnp.int32),
            pltpu.VMEM((3 * bpw // _CHUNK, _CHUNK), jnp.int32),
            pltpu.VMEM((3 * bpw,), jnp.int32),
            pltpu.VMEM((3 * bpw * dim // _CHUNK, _CHUNK), jnp.int32),
            pltpu.VMEM((3 * bpw * dim,), jnp.float32),
            pltpu.SemaphoreType.DMA,
        ],
    )
    out = run(lab3, uids, embf)
    return out.reshape(batch, 3 * dim)


# speedup vs baseline: 5.5355x; 1.3034x over previous
import functools

import jax
import jax.numpy as jnp
from jax import lax
from jax.experimental import pallas as pl
from jax.experimental.pallas import tpu as pltpu
from jax.experimental.pallas import tpu_sc as plsc

_NUM_USERS = 100000
_VOCAB = 1000000
_NC, _NS = 2, 16
_NW = _NC * _NS
_CHUNK = 128
_L = 16
_FIRE = 24


def _body(lab3_hbm, uids_hbm, embf_hbm, out_hbm,
          uids_v, idx1_v, ids_v, int_v, idx2_v, emb_v, sem, bpw, dim):
    wid = lax.axis_index("s") * _NC + lax.axis_index("c")
    base = wid * bpw
    n3 = 3 * bpw
    nel = n3 * dim

    pltpu.sync_copy(uids_hbm.at[pl.ds(base, bpw)], uids_v)

    for j in range(3):
        for k in range(bpw // _L):
            p = j * bpw + k * _L
            u = uids_v[pl.ds(k * _L, _L)]
            idx1_v[p // _CHUNK, pl.ds(p % _CHUNK, _L)] = u + j * _NUM_USERS

    cps = [pltpu.async_copy(lab3_hbm.at[idx1_v.at[i]], ids_v.at[i], sem)
           for i in range(n3 // _CHUNK)]
    for cp in cps:
        cp.wait()

    three = jnp.full((_L,), 3, jnp.int32)
    iota = lax.iota(jnp.int32, _L)
    for k in range(n3 // _L):
        p = k * _L
        pos = iota + p
        b = lax.div(pos, three)
        j = lax.rem(pos, three)
        q = j * bpw + b
        qr = lax.shift_right_logical(q, 7)
        qc = lax.bitwise_and(q, _CHUNK - 1)
        int_v[pl.ds(p, _L)] = plsc.load_gather(ids_v, [qr, qc])

    wvec = jnp.full((_L,), _WCH, jnp.int32)
    dconst = (lax.shift_right_logical(iota, 3) * (8 * _WCH)
              + lax.bitwise_and(iota, 7) * 128)

    def expand(k, _):
        e = plsc.load_gather(int_v, [jnp.full((_L,), k, jnp.int32)])
        ch = lax.div(e, wvec)
        r = lax.rem(e, wvec)
        idx2_v[lax.div(k, 8), pl.ds(lax.rem(k, 8) * _L, _L)] = (
            ch * (16 * _WCH)
            + lax.shift_left(lax.shift_right_logical(r, 7), 10)
            + lax.bitwise_and(r, 127)
            + dconst)
        return _

    lax.fori_loop(0, n3, expand, 0)

    nch = nel // _CHUNK
    for g in range(0, nch, _FIRE):
        cps = [pltpu.async_copy(embf_hbm.at[idx2_v.at[i]],
                                emb_v.at[pl.ds(i * _CHUNK, _CHUNK)], sem)
               for i in range(g, min(g + _FIRE, nch))]
        for cp in cps:
            cp.wait()

    pltpu.sync_copy(emb_v, out_hbm.at[pl.ds(wid * nel, nel)])


_WCH = 15616


def _retile_body(in_ref, out_ref):
    out_ref[...] = (in_ref[...]
                    .reshape(8, _WCH // 128, 128)
                    .swapaxes(0, 1)
                    .reshape(8 * _WCH // 128, 128))


def _retile_tc(emb_t):
    dim, vocab = emb_t.shape
    nch = (vocab + _WCH - 1) // _WCH
    rows = 8 * _WCH // 128
    return pl.pallas_call(
        _retile_body,
        grid=(dim // 8, nch),
        in_specs=[pl.BlockSpec((8, _WCH), lambda g, c: (g, c))],
        out_specs=pl.BlockSpec((rows, 128), lambda g, c: (c * (dim // 8) + g, 0)),
        out_shape=jax.ShapeDtypeStruct(((dim // 8) * nch * rows, 128),
                                       jnp.float32),
    )(emb_t)


@jax.jit
def kernel(uids, train_labels, item_embeddings):
    batch = uids.shape[0]
    dim = item_embeddings.shape[1]
    hist = train_labels.shape[1]
    bpw = batch // _NW
    lab3 = train_labels.T[hist - 3:hist].reshape(-1)
    embf = _retile_tc(item_embeddings.T).reshape(-1)

    run = pl.kernel(
        functools.partial(_body, bpw=bpw, dim=dim),
        out_type=jax.ShapeDtypeStruct((3 * batch * dim,), jnp.float32),
        mesh=plsc.VectorSubcoreMesh(core_axis_name="c", subcore_axis_name="s"),
        compiler_params=pltpu.CompilerParams(
            needs_layout_passes=False, use_tc_tiling_on_sc=False),
        scratch_types=[
            pltpu.VMEM((bpw,), jnp.int32),
            pltpu.VMEM((3 * bpw // _CHUNK, _CHUNK), jnp.int32),
            pltpu.VMEM((3 * bpw // _CHUNK, _CHUNK), jnp.int32),
            pltpu.VMEM((3 * bpw,), jnp.int32),
            pltpu.VMEM((3 * bpw * dim // _CHUNK, _CHUNK), jnp.int32),
            pltpu.VMEM((3 * bpw * dim,), jnp.float32),
            pltpu.SemaphoreType.DMA,
        ],
    )
    out = run(lab3, uids, embf)
    return out.reshape(batch, 3 * dim)


# speedup vs baseline: 5.6673x; 1.0238x over previous
import functools

import jax
import jax.numpy as jnp
from jax import lax
from jax.experimental import pallas as pl
from jax.experimental.pallas import tpu as pltpu
from jax.experimental.pallas import tpu_sc as plsc

_NUM_USERS = 100000
_VOCAB = 1000000
_NC, _NS = 2, 16
_NW = _NC * _NS
_CHUNK = 128
_L = 16
_FIRE = 24


def _body(lab3_hbm, uids_hbm, embf_hbm, out_hbm,
          uids_v, idx1_v, ids_v, int_v, idx2_v, emb_v, sem, bpw, dim):
    wid = lax.axis_index("s") * _NC + lax.axis_index("c")
    base = wid * bpw
    n3 = 3 * bpw
    nel = n3 * dim

    pltpu.sync_copy(uids_hbm.at[pl.ds(base, bpw)], uids_v)

    for j in range(3):
        for k in range(bpw // _L):
            p = j * bpw + k * _L
            u = uids_v[pl.ds(k * _L, _L)]
            idx1_v[pl.ds(p, _L)] = u + j * _NUM_USERS

    pltpu.async_copy(lab3_hbm.at[idx1_v], ids_v, sem).wait()

    three = jnp.full((_L,), 3, jnp.int32)
    iota = lax.iota(jnp.int32, _L)
    for k in range(n3 // _L):
        p = k * _L
        pos = iota + p
        b = lax.div(pos, three)
        j = lax.rem(pos, three)
        int_v[pl.ds(p, _L)] = plsc.load_gather(ids_v, [j * bpw + b])

    wvec = jnp.full((_L,), _WCH, jnp.int32)
    dconst = (lax.shift_right_logical(iota, 3) * (8 * _WCH)
              + lax.bitwise_and(iota, 7) * 128)

    def expand(k, _):
        e = plsc.load_gather(int_v, [jnp.full((_L,), k, jnp.int32)])
        ch = lax.div(e, wvec)
        r = lax.rem(e, wvec)
        idx2_v[pl.ds(k * _L, _L)] = (
            ch * (16 * _WCH)
            + lax.shift_left(lax.shift_right_logical(r, 7), 10)
            + lax.bitwise_and(r, 127)
            + dconst)
        return _

    lax.fori_loop(0, n3, expand, 0)

    pltpu.async_copy(embf_hbm.at[idx2_v], emb_v, sem).wait()

    pltpu.sync_copy(emb_v, out_hbm.at[pl.ds(wid * nel, nel)])


_WCH = 15616


def _retile_body(in_ref, out_ref):
    out_ref[...] = (in_ref[...]
                    .reshape(8, _WCH // 128, 128)
                    .swapaxes(0, 1)
                    .reshape(8 * _WCH // 128, 128))


def _retile_tc(emb_t):
    dim, vocab = emb_t.shape
    nch = (vocab + _WCH - 1) // _WCH
    rows = 8 * _WCH // 128
    return pl.pallas_call(
        _retile_body,
        grid=(dim // 8, nch),
        in_specs=[pl.BlockSpec((8, _WCH), lambda g, c: (g, c))],
        out_specs=pl.BlockSpec((rows, 128), lambda g, c: (c * (dim // 8) + g, 0)),
        out_shape=jax.ShapeDtypeStruct(((dim // 8) * nch * rows, 128),
                                       jnp.float32),
    )(emb_t)


@jax.jit
def kernel(uids, train_labels, item_embeddings):
    batch = uids.shape[0]
    dim = item_embeddings.shape[1]
    hist = train_labels.shape[1]
    bpw = batch // _NW
    lab3 = train_labels.T[hist - 3:hist].reshape(-1)
    embf = _retile_tc(item_embeddings.T).reshape(-1)

    run = pl.kernel(
        functools.partial(_body, bpw=bpw, dim=dim),
        out_type=jax.ShapeDtypeStruct((3 * batch * dim,), jnp.float32),
        mesh=plsc.VectorSubcoreMesh(core_axis_name="c", subcore_axis_name="s"),
        compiler_params=pltpu.CompilerParams(
            needs_layout_passes=False, use_tc_tiling_on_sc=False),
        scratch_types=[
            pltpu.VMEM((bpw,), jnp.int32),
            pltpu.VMEM((3 * bpw,), jnp.int32),
            pltpu.VMEM((3 * bpw,), jnp.int32),
            pltpu.VMEM((3 * bpw,), jnp.int32),
            pltpu.VMEM((3 * bpw * dim,), jnp.int32),
            pltpu.VMEM((3 * bpw * dim,), jnp.float32),
            pltpu.SemaphoreType.DMA,
        ],
    )
    out = run(lab3, uids, embf)
    return out.reshape(batch, 3 * dim)


# speedup vs baseline: 6.3938x; 1.1282x over previous
import functools

import jax
import jax.numpy as jnp
from jax import lax
from jax.experimental import pallas as pl
from jax.experimental.pallas import tpu as pltpu
from jax.experimental.pallas import tpu_sc as plsc

_NUM_USERS = 100000
_VOCAB = 1000000
_NC, _NS = 2, 16
_NW = _NC * _NS
_CHUNK = 128
_L = 16
_FIRE = 24


def _body(lab3_hbm, uids_hbm, embf_hbm, out_hbm,
          uids_v, idx1_v, ids_v, int_v, idx2_v, emb_v, sem, bpw, dim):
    wid = lax.axis_index("s") * _NC + lax.axis_index("c")
    base = wid * bpw
    n3 = 3 * bpw
    nel = n3 * dim

    pltpu.sync_copy(uids_hbm.at[pl.ds(base, bpw)], uids_v)

    for j in range(3):
        for k in range(bpw // _L):
            p = j * bpw + k * _L
            u = uids_v[pl.ds(k * _L, _L)]
            idx1_v[pl.ds(p, _L)] = u + j * _NUM_USERS

    pltpu.async_copy(lab3_hbm.at[idx1_v], ids_v, sem).wait()

    three = jnp.full((_L,), 3, jnp.int32)
    iota = lax.iota(jnp.int32, _L)
    for k in range(n3 // _L):
        p = k * _L
        pos = iota + p
        b = lax.div(pos, three)
        j = lax.rem(pos, three)
        int_v[pl.ds(p, _L)] = plsc.load_gather(ids_v, [j * bpw + b])

    wvec = jnp.full((_L,), _WCH, jnp.int32)
    dconst = (lax.shift_right_logical(iota, 3) * (8 * _WCH)
              + lax.bitwise_and(iota, 7) * 128)

    def expand(k, _):
        e = plsc.load_gather(int_v, [jnp.full((_L,), k, jnp.int32)])
        ch = lax.div(e, wvec)
        r = lax.rem(e, wvec)
        idx2_v[pl.ds(k * _L, _L)] = (
            ch * (16 * _WCH)
            + lax.shift_left(lax.shift_right_logical(r, 7), 10)
            + lax.bitwise_and(r, 127)
            + dconst)
        return _

    lax.fori_loop(0, n3, expand, 0)

    pltpu.async_copy(embf_hbm.at[idx2_v], emb_v, sem).wait()

    pltpu.sync_copy(emb_v, out_hbm.at[pl.ds(wid * nel, nel)])


_WCH = 31232


def _retile_body(in_ref, out_ref):
    out_ref[...] = (in_ref[...]
                    .reshape(8, _WCH // 128, 128)
                    .swapaxes(0, 1)
                    .reshape(8 * _WCH // 128, 128))


def _retile_tc(emb_t):
    dim, vocab = emb_t.shape
    nch = (vocab + _WCH - 1) // _WCH
    rows = 8 * _WCH // 128
    return pl.pallas_call(
        _retile_body,
        grid=(dim // 8, nch),
        in_specs=[pl.BlockSpec((8, _WCH), lambda g, c: (g, c))],
        out_specs=pl.BlockSpec((rows, 128), lambda g, c: (c * (dim // 8) + g, 0)),
        out_shape=jax.ShapeDtypeStruct(((dim // 8) * nch * rows, 128),
                                       jnp.float32),
    )(emb_t)


@jax.jit
def kernel(uids, train_labels, item_embeddings):
    batch = uids.shape[0]
    dim = item_embeddings.shape[1]
    hist = train_labels.shape[1]
    bpw = batch // _NW
    lab3 = train_labels.T[hist - 3:hist].reshape(-1)
    embf = _retile_tc(item_embeddings.T).reshape(-1)

    run = pl.kernel(
        functools.partial(_body, bpw=bpw, dim=dim),
        out_type=jax.ShapeDtypeStruct((3 * batch * dim,), jnp.float32),
        mesh=plsc.VectorSubcoreMesh(core_axis_name="c", subcore_axis_name="s"),
        compiler_params=pltpu.CompilerParams(
            needs_layout_passes=False, use_tc_tiling_on_sc=False),
        scratch_types=[
            pltpu.VMEM((bpw,), jnp.int32),
            pltpu.VMEM((3 * bpw,), jnp.int32),
            pltpu.VMEM((3 * bpw,), jnp.int32),
            pltpu.VMEM((3 * bpw,), jnp.int32),
            pltpu.VMEM((3 * bpw * dim,), jnp.int32),
            pltpu.VMEM((3 * bpw * dim,), jnp.float32),
            pltpu.SemaphoreType.DMA,
        ],
    )
    out = run(lab3, uids, embf)
    return out.reshape(batch, 3 * dim)


# speedup vs baseline: 7.0239x; 1.0985x over previous
import functools

import jax
import jax.numpy as jnp
from jax import lax
from jax.experimental import pallas as pl
from jax.experimental.pallas import tpu as pltpu
from jax.experimental.pallas import tpu_sc as plsc

_NUM_USERS = 100000
_VOCAB = 1000000
_NC, _NS = 2, 16
_NW = _NC * _NS
_CHUNK = 128
_L = 16
_FIRE = 24


def _body(lab3_hbm, uids_hbm, embf_hbm, out_hbm,
          uids_v, idx1_v, ids_v, int_v, idx2_v, emb_v, sem, bpw, dim):
    wid = lax.axis_index("s") * _NC + lax.axis_index("c")
    base = wid * bpw
    n3 = 3 * bpw
    nel = n3 * dim

    pltpu.sync_copy(uids_hbm.at[pl.ds(base, bpw)], uids_v)

    for j in range(3):
        for k in range(bpw // _L):
            p = j * bpw + k * _L
            u = uids_v[pl.ds(k * _L, _L)]
            idx1_v[pl.ds(p, _L)] = u + j * _NUM_USERS

    pltpu.async_copy(lab3_hbm.at[idx1_v], ids_v, sem).wait()

    three = jnp.full((_L,), 3, jnp.int32)
    iota = lax.iota(jnp.int32, _L)
    for k in range(n3 // _L):
        p = k * _L
        pos = iota + p
        b = lax.div(pos, three)
        j = lax.rem(pos, three)
        int_v[pl.ds(p, _L)] = plsc.load_gather(ids_v, [j * bpw + b])

    wvec = jnp.full((_L,), _WCH, jnp.int32)
    dconst = (lax.shift_right_logical(iota, 3) * (8 * _WCH)
              + lax.bitwise_and(iota, 7) * 128)

    def expand(k, _):
        e = plsc.load_gather(int_v, [jnp.full((_L,), k, jnp.int32)])
        ch = lax.div(e, wvec)
        r = lax.rem(e, wvec)
        idx2_v[pl.ds(k * _L, _L)] = (
            ch * (16 * _WCH)
            + lax.shift_left(lax.shift_right_logical(r, 7), 10)
            + lax.bitwise_and(r, 127)
            + dconst)
        return _

    lax.fori_loop(0, n3, expand, 0)

    pltpu.async_copy(embf_hbm.at[idx2_v], emb_v, sem).wait()

    pltpu.sync_copy(emb_v, out_hbm.at[pl.ds(wid * nel, nel)])


_WCH = 62464


def _retile_body(in_ref, out_ref):
    out_ref[...] = (in_ref[...]
                    .reshape(8, _WCH // 128, 128)
                    .swapaxes(0, 1)
                    .reshape(8 * _WCH // 128, 128))


def _retile_tc(emb_t):
    dim, vocab = emb_t.shape
    nch = (vocab + _WCH - 1) // _WCH
    rows = 8 * _WCH // 128
    return pl.pallas_call(
        _retile_body,
        grid=(dim // 8, nch),
        in_specs=[pl.BlockSpec((8, _WCH), lambda g, c: (g, c))],
        out_specs=pl.BlockSpec((rows, 128), lambda g, c: (c * (dim // 8) + g, 0)),
        out_shape=jax.ShapeDtypeStruct(((dim // 8) * nch * rows, 128),
                                       jnp.float32),
    )(emb_t)


@jax.jit
def kernel(uids, train_labels, item_embeddings):
    batch = uids.shape[0]
    dim = item_embeddings.shape[1]
    hist = train_labels.shape[1]
    bpw = batch // _NW
    lab3 = train_labels.T[hist - 3:hist].reshape(-1)
    embf = _retile_tc(item_embeddings.T).reshape(-1)

    run = pl.kernel(
        functools.partial(_body, bpw=bpw, dim=dim),
        out_type=jax.ShapeDtypeStruct((3 * batch * dim,), jnp.float32),
        mesh=plsc.VectorSubcoreMesh(core_axis_name="c", subcore_axis_name="s"),
        compiler_params=pltpu.CompilerParams(
            needs_layout_passes=False, use_tc_tiling_on_sc=False),
        scratch_types=[
            pltpu.VMEM((bpw,), jnp.int32),
            pltpu.VMEM((3 * bpw,), jnp.int32),
            pltpu.VMEM((3 * bpw,), jnp.int32),
            pltpu.VMEM((3 * bpw,), jnp.int32),
            pltpu.VMEM((3 * bpw * dim,), jnp.int32),
            pltpu.VMEM((3 * bpw * dim,), jnp.float32),
            pltpu.SemaphoreType.DMA,
        ],
    )
    out = run(lab3, uids, embf)
    return out.reshape(batch, 3 * dim)


# speedup vs baseline: 7.1968x; 1.0246x over previous
import functools

import jax
import jax.numpy as jnp
from jax import lax
from jax.experimental import pallas as pl
from jax.experimental.pallas import tpu as pltpu
from jax.experimental.pallas import tpu_sc as plsc

_NUM_USERS = 100000
_VOCAB = 1000000
_NC, _NS = 2, 16
_NW = _NC * _NS
_CHUNK = 128
_L = 16
_FIRE = 24


def _body(lab3_hbm, uids_hbm, embf_hbm, out_hbm,
          uids_v, idx1_v, ids_v, int_v, idx2_v, emb_v, sem, bpw, dim):
    wid = lax.axis_index("s") * _NC + lax.axis_index("c")
    base = wid * bpw
    n3 = 3 * bpw
    nel = n3 * dim

    pltpu.sync_copy(uids_hbm.at[pl.ds(base, bpw)], uids_v)

    for j in range(3):
        for k in range(bpw // _L):
            p = j * bpw + k * _L
            u = uids_v[pl.ds(k * _L, _L)]
            idx1_v[pl.ds(p, _L)] = u + j * _NUM_USERS

    pltpu.async_copy(lab3_hbm.at[idx1_v], ids_v, sem).wait()

    three = jnp.full((_L,), 3, jnp.int32)
    iota = lax.iota(jnp.int32, _L)
    for k in range(n3 // _L):
        p = k * _L
        pos = iota + p
        b = lax.div(pos, three)
        j = lax.rem(pos, three)
        int_v[pl.ds(p, _L)] = plsc.load_gather(ids_v, [j * bpw + b])

    wvec = jnp.full((_L,), _WCH, jnp.int32)
    dconst = (lax.shift_right_logical(iota, 3) * (8 * _WCH)
              + lax.bitwise_and(iota, 7) * 128)

    def expand(k, _):
        e = plsc.load_gather(int_v, [jnp.full((_L,), k, jnp.int32)])
        ch = lax.div(e, wvec)
        r = lax.rem(e, wvec)
        idx2_v[pl.ds(k * _L, _L)] = (
            ch * (16 * _WCH)
            + lax.shift_left(lax.shift_right_logical(r, 7), 10)
            + lax.bitwise_and(r, 127)
            + dconst)
        return _

    lax.fori_loop(0, n3, expand, 0)

    pltpu.async_copy(embf_hbm.at[idx2_v], emb_v, sem).wait()

    pltpu.sync_copy(emb_v, out_hbm.at[pl.ds(wid * nel, nel)])


_WCH = 124928


def _retile_body(in_ref, out_ref):
    out_ref[...] = (in_ref[...]
                    .reshape(8, _WCH // 128, 128)
                    .swapaxes(0, 1)
                    .reshape(8 * _WCH // 128, 128))


def _retile_tc(emb_t):
    dim, vocab = emb_t.shape
    nch = (vocab + _WCH - 1) // _WCH
    rows = 8 * _WCH // 128
    return pl.pallas_call(
        _retile_body,
        grid=(dim // 8, nch),
        in_specs=[pl.BlockSpec((8, _WCH), lambda g, c: (g, c))],
        out_specs=pl.BlockSpec((rows, 128), lambda g, c: (c * (dim // 8) + g, 0)),
        out_shape=jax.ShapeDtypeStruct(((dim // 8) * nch * rows, 128),
                                       jnp.float32),
    )(emb_t)


@jax.jit
def kernel(uids, train_labels, item_embeddings):
    batch = uids.shape[0]
    dim = item_embeddings.shape[1]
    hist = train_labels.shape[1]
    bpw = batch // _NW
    lab3 = train_labels.T[hist - 3:hist].reshape(-1)
    embf = _retile_tc(item_embeddings.T).reshape(-1)

    run = pl.kernel(
        functools.partial(_body, bpw=bpw, dim=dim),
        out_type=jax.ShapeDtypeStruct((3 * batch * dim,), jnp.float32),
        mesh=plsc.VectorSubcoreMesh(core_axis_name="c", subcore_axis_name="s"),
        compiler_params=pltpu.CompilerParams(
            needs_layout_passes=False, use_tc_tiling_on_sc=False),
        scratch_types=[
            pltpu.VMEM((bpw,), jnp.int32),
            pltpu.VMEM((3 * bpw,), jnp.int32),
            pltpu.VMEM((3 * bpw,), jnp.int32),
            pltpu.VMEM((3 * bpw,), jnp.int32),
            pltpu.VMEM((3 * bpw * dim,), jnp.int32),
            pltpu.VMEM((3 * bpw * dim,), jnp.float32),
            pltpu.SemaphoreType.DMA,
        ],
    )
    out = run(lab3, uids, embf)
    return out.reshape(batch, 3 * dim)


# speedup vs baseline: 9.0318x; 1.2550x over previous
import functools

import jax
import jax.numpy as jnp
from jax import lax
from jax.experimental import pallas as pl
from jax.experimental.pallas import tpu as pltpu
from jax.experimental.pallas import tpu_sc as plsc

_NUM_USERS = 100000
_NC, _NS = 2, 16
_NW = _NC * _NS
_L = 16
_WCH = 124928
_UNROLL = 8


def _idx_body(lab3_hbm, uids_hbm, idx2_hbm,
              uids_v, idx1_v, ids_v, int_v, idx2_v, sem, bpw, dim):
    wid = lax.axis_index("s") * _NC + lax.axis_index("c")
    base = wid * bpw
    n3 = 3 * bpw
    nel = n3 * dim

    pltpu.sync_copy(uids_hbm.at[pl.ds(base, bpw)], uids_v)

    for j in range(3):
        for k in range(bpw // _L):
            p = j * bpw + k * _L
            u = uids_v[pl.ds(k * _L, _L)]
            idx1_v[pl.ds(p, _L)] = u + j * _NUM_USERS

    pltpu.async_copy(lab3_hbm.at[idx1_v], ids_v, sem).wait()

    three = jnp.full((_L,), 3, jnp.int32)
    iota = lax.iota(jnp.int32, _L)
    for k in range(n3 // _L):
        pos = iota + k * _L
        b = lax.div(pos, three)
        j = lax.rem(pos, three)
        int_v[pl.ds(k * _L, _L)] = plsc.load_gather(ids_v, [j * bpw + b])

    wvec = jnp.full((_L,), _WCH, jnp.int32)
    dconst = (lax.shift_right_logical(iota, 3) * (8 * _WCH)
              + lax.bitwise_and(iota, 7) * 128)

    def expand(k, _):
        for m in range(_UNROLL):
            km = k * _UNROLL + m
            e = plsc.load_gather(int_v, [jnp.full((_L,), km, jnp.int32)])
            ch = lax.div(e, wvec)
            r = lax.rem(e, wvec)
            idx2_v[pl.ds(km * _L, _L)] = (
                ch * (16 * _WCH)
                + lax.shift_left(lax.shift_right_logical(r, 7), 10)
                + lax.bitwise_and(r, 127)
                + dconst)
        return _

    lax.fori_loop(0, n3 // _UNROLL, expand, 0)

    pltpu.sync_copy(idx2_v, idx2_hbm.at[pl.ds(wid * nel, nel)])


def _gather_body(embf_hbm, idx2_hbm, out_hbm, idx2_v, emb_v, sem, nel):
    wid = lax.axis_index("s") * _NC + lax.axis_index("c")
    pltpu.sync_copy(idx2_hbm.at[pl.ds(wid * nel, nel)], idx2_v)
    pltpu.async_copy(embf_hbm.at[idx2_v], emb_v, sem).wait()
    pltpu.sync_copy(emb_v, out_hbm.at[pl.ds(wid * nel, nel)])


def _retile_body(in_ref, out_ref):
    out_ref[...] = (in_ref[...]
                    .reshape(8, _WCH // 128, 128)
                    .swapaxes(0, 1)
                    .reshape(8 * _WCH // 128, 128))


def _retile_tc(emb_t):
    dim, vocab = emb_t.shape
    nch = (vocab + _WCH - 1) // _WCH
    rows = 8 * _WCH // 128
    return pl.pallas_call(
        _retile_body,
        grid=(dim // 8, nch),
        in_specs=[pl.BlockSpec((8, _WCH), lambda g, c: (g, c))],
        out_specs=pl.BlockSpec((rows, 128), lambda g, c: (c * (dim // 8) + g, 0)),
        out_shape=jax.ShapeDtypeStruct(((dim // 8) * nch * rows, 128),
                                       jnp.float32),
    )(emb_t)


@jax.jit
def kernel(uids, train_labels, item_embeddings):
    batch = uids.shape[0]
    dim = item_embeddings.shape[1]
    hist = train_labels.shape[1]
    bpw = batch // _NW
    nel = 3 * bpw * dim
    sc_params = pltpu.CompilerParams(
        needs_layout_passes=False, use_tc_tiling_on_sc=False)
    mesh = plsc.VectorSubcoreMesh(core_axis_name="c", subcore_axis_name="s")

    lab3 = train_labels.T[hist - 3:hist].reshape(-1)
    embf = _retile_tc(item_embeddings.T).reshape(-1)

    idx2 = pl.kernel(
        functools.partial(_idx_body, bpw=bpw, dim=dim),
        out_type=jax.ShapeDtypeStruct((3 * batch * dim,), jnp.int32),
        mesh=mesh,
        compiler_params=sc_params,
        scratch_types=[
            pltpu.VMEM((bpw,), jnp.int32),
            pltpu.VMEM((3 * bpw,), jnp.int32),
            pltpu.VMEM((3 * bpw,), jnp.int32),
            pltpu.VMEM((3 * bpw,), jnp.int32),
            pltpu.VMEM((nel,), jnp.int32),
            pltpu.SemaphoreType.DMA,
        ],
    )(lab3, uids)

    out = pl.kernel(
        functools.partial(_gather_body, nel=nel),
        out_type=jax.ShapeDtypeStruct((3 * batch * dim,), jnp.float32),
        mesh=mesh,
        compiler_params=sc_params,
        scratch_types=[
            pltpu.VMEM((nel,), jnp.int32),
            pltpu.VMEM((nel,), jnp.float32),
            pltpu.SemaphoreType.DMA,
        ],
    )(embf, idx2)
    return out.reshape(batch, 3 * dim)


# speedup vs baseline: 11.5547x; 1.2793x over previous
import functools

import jax
import jax.numpy as jnp
from jax import lax
from jax.experimental import pallas as pl
from jax.experimental.pallas import tpu as pltpu
from jax.experimental.pallas import tpu_sc as plsc

_NUM_USERS = 100000
_NC, _NS = 2, 16
_NW = _NC * _NS
_L = 16
_WCH = 124928
_UNROLL = 8


def _idx_body(lab3_hbm, uids_hbm, idx2_hbm,
              uids_v, idx1_v, ids_v, int_v, idx2_v, sem, bpw, dim):
    wid = lax.axis_index("s") * _NC + lax.axis_index("c")
    base = wid * bpw
    n3 = 3 * bpw
    nel = n3 * dim

    pltpu.sync_copy(uids_hbm.at[pl.ds(base, bpw)], uids_v)

    for j in range(3):
        for k in range(bpw // _L):
            p = j * bpw + k * _L
            u = uids_v[pl.ds(k * _L, _L)]
            idx1_v[pl.ds(p, _L)] = u + j * _NUM_USERS

    pltpu.async_copy(lab3_hbm.at[idx1_v], ids_v, sem).wait()

    three = jnp.full((_L,), 3, jnp.int32)
    iota = lax.iota(jnp.int32, _L)
    wvec = jnp.full((_L,), _WCH, jnp.int32)
    dconst = (lax.shift_right_logical(iota, 3) * (8 * _WCH)
              + lax.bitwise_and(iota, 7) * 128)

    def expand(k, _):
        pos = iota + k * _L
        b = lax.div(pos, three)
        j = lax.rem(pos, three)
        e = plsc.load_gather(ids_v, [j * bpw + b])
        ch = lax.div(e, wvec)
        r = lax.rem(e, wvec)
        int_v[pl.ds(0, _L)] = (
            ch * (16 * _WCH)
            + lax.shift_left(lax.shift_right_logical(r, 7), 10)
            + lax.bitwise_and(r, 127))
        for m in range(_L):
            sv = plsc.load_gather(int_v, [jnp.full((_L,), m, jnp.int32)])
            idx2_v[pl.ds(k * _L * _L + m * _L, _L)] = sv + dconst
        return _

    lax.fori_loop(0, n3 // _L, expand, 0)

    pltpu.sync_copy(idx2_v, idx2_hbm.at[pl.ds(wid * nel, nel)])


def _gather_body(embf_hbm, idx2_hbm, out_hbm, idx2_v, emb_v, sem, nel):
    wid = lax.axis_index("s") * _NC + lax.axis_index("c")
    pltpu.sync_copy(idx2_hbm.at[pl.ds(wid * nel, nel)], idx2_v)
    pltpu.async_copy(embf_hbm.at[idx2_v], emb_v, sem).wait()
    pltpu.sync_copy(emb_v, out_hbm.at[pl.ds(wid * nel, nel)])


def _retile_body(in_ref, out_ref):
    out_ref[...] = (in_ref[...]
                    .reshape(8, _WCH // 128, 128)
                    .swapaxes(0, 1)
                    .reshape(8 * _WCH // 128, 128))


def _retile_tc(emb_t):
    dim, vocab = emb_t.shape
    nch = (vocab + _WCH - 1) // _WCH
    rows = 8 * _WCH // 128
    return pl.pallas_call(
        _retile_body,
        grid=(dim // 8, nch),
        in_specs=[pl.BlockSpec((8, _WCH), lambda g, c: (g, c))],
        out_specs=pl.BlockSpec((rows, 128), lambda g, c: (c * (dim // 8) + g, 0)),
        out_shape=jax.ShapeDtypeStruct(((dim // 8) * nch * rows, 128),
                                       jnp.float32),
    )(emb_t)


@jax.jit
def kernel(uids, train_labels, item_embeddings):
    batch = uids.shape[0]
    dim = item_embeddings.shape[1]
    hist = train_labels.shape[1]
    bpw = batch // _NW
    nel = 3 * bpw * dim
    sc_params = pltpu.CompilerParams(
        needs_layout_passes=False, use_tc_tiling_on_sc=False)
    mesh = plsc.VectorSubcoreMesh(core_axis_name="c", subcore_axis_name="s")

    lab3 = train_labels.T[hist - 3:hist].reshape(-1)
    embf = _retile_tc(item_embeddings.T).reshape(-1)

    idx2 = pl.kernel(
        functools.partial(_idx_body, bpw=bpw, dim=dim),
        out_type=jax.ShapeDtypeStruct((3 * batch * dim,), jnp.int32),
        mesh=mesh,
        compiler_params=sc_params,
        scratch_types=[
            pltpu.VMEM((bpw,), jnp.int32),
            pltpu.VMEM((3 * bpw,), jnp.int32),
            pltpu.VMEM((3 * bpw,), jnp.int32),
            pltpu.VMEM((3 * bpw,), jnp.int32),
            pltpu.VMEM((nel,), jnp.int32),
            pltpu.SemaphoreType.DMA,
        ],
    )(lab3, uids)

    out = pl.kernel(
        functools.partial(_gather_body, nel=nel),
        out_type=jax.ShapeDtypeStruct((3 * batch * dim,), jnp.float32),
        mesh=mesh,
        compiler_params=sc_params,
        scratch_types=[
            pltpu.VMEM((nel,), jnp.int32),
            pltpu.VMEM((nel,), jnp.float32),
            pltpu.SemaphoreType.DMA,
        ],
    )(embf, idx2)
    return out.reshape(batch, 3 * dim)


# speedup vs baseline: 11.5581x; 1.0003x over previous
import functools

import jax
import jax.numpy as jnp
from jax import lax
from jax.experimental import pallas as pl
from jax.experimental.pallas import tpu as pltpu
from jax.experimental.pallas import tpu_sc as plsc

_NUM_USERS = 100000
_NC, _NS = 2, 16
_NW = _NC * _NS
_L = 16
_WCH = 124928
_UNROLL = 8


def _idx_body(lab3_hbm, uids_hbm, idx2_hbm,
              uids_v, idx1_v, ids_v, int_v, idx2_v, sem, bpw, dim):
    wid = lax.axis_index("s") * _NC + lax.axis_index("c")
    base = wid * bpw
    n3 = 3 * bpw
    nel = n3 * dim

    pltpu.sync_copy(uids_hbm.at[pl.ds(base, bpw)], uids_v)

    for j in range(3):
        for k in range(bpw // _L):
            p = j * bpw + k * _L
            u = uids_v[pl.ds(k * _L, _L)]
            idx1_v[pl.ds(p, _L)] = u + j * _NUM_USERS

    pltpu.async_copy(lab3_hbm.at[idx1_v], ids_v, sem).wait()

    three = jnp.full((_L,), 3, jnp.int32)
    iota = lax.iota(jnp.int32, _L)
    wvec = jnp.full((_L,), _WCH, jnp.int32)
    dconst = (lax.shift_right_logical(iota, 3) * (8 * _WCH)
              + lax.bitwise_and(iota, 7) * 128)

    def expand(k, _):
        pos = iota + k * _L
        b = lax.div(pos, three)
        j = lax.rem(pos, three)
        e = plsc.load_gather(ids_v, [j * bpw + b])
        ch = lax.div(e, wvec)
        r = lax.rem(e, wvec)
        s = (ch * (16 * _WCH)
             + lax.shift_left(lax.shift_right_logical(r, 7), 10)
             + lax.bitwise_and(r, 127))
        for m in range(_L):
            sv = lax.gather(
                s, jnp.full((_L, 1), m, jnp.int32),
                lax.GatherDimensionNumbers(
                    offset_dims=(), collapsed_slice_dims=(0,),
                    start_index_map=(0,)),
                (1,), mode=lax.GatherScatterMode.PROMISE_IN_BOUNDS)
            idx2_v[pl.ds(k * _L * _L + m * _L, _L)] = sv + dconst
        return _

    lax.fori_loop(0, n3 // _L, expand, 0)

    pltpu.sync_copy(idx2_v, idx2_hbm.at[pl.ds(wid * nel, nel)])


def _gather_body(embf_hbm, idx2_hbm, out_hbm, idx2_v, emb_v, sem, nel):
    wid = lax.axis_index("s") * _NC + lax.axis_index("c")
    pltpu.sync_copy(idx2_hbm.at[pl.ds(wid * nel, nel)], idx2_v)
    pltpu.async_copy(embf_hbm.at[idx2_v], emb_v, sem).wait()
    pltpu.sync_copy(emb_v, out_hbm.at[pl.ds(wid * nel, nel)])


def _retile_body(in_ref, out_ref):
    out_ref[...] = (in_ref[...]
                    .reshape(8, _WCH // 128, 128)
                    .swapaxes(0, 1)
                    .reshape(8 * _WCH // 128, 128))


def _retile_tc(emb_t):
    dim, vocab = emb_t.shape
    nch = (vocab + _WCH - 1) // _WCH
    rows = 8 * _WCH // 128
    return pl.pallas_call(
        _retile_body,
        grid=(dim // 8, nch),
        in_specs=[pl.BlockSpec((8, _WCH), lambda g, c: (g, c))],
        out_specs=pl.BlockSpec((rows, 128), lambda g, c: (c * (dim // 8) + g, 0)),
        out_shape=jax.ShapeDtypeStruct(((dim // 8) * nch * rows, 128),
                                       jnp.float32),
    )(emb_t)


@jax.jit
def kernel(uids, train_labels, item_embeddings):
    batch = uids.shape[0]
    dim = item_embeddings.shape[1]
    hist = train_labels.shape[1]
    bpw = batch // _NW
    nel = 3 * bpw * dim
    sc_params = pltpu.CompilerParams(
        needs_layout_passes=False, use_tc_tiling_on_sc=False)
    mesh = plsc.VectorSubcoreMesh(core_axis_name="c", subcore_axis_name="s")

    lab3 = train_labels.T[hist - 3:hist].reshape(-1)
    embf = _retile_tc(item_embeddings.T).reshape(-1)

    idx2 = pl.kernel(
        functools.partial(_idx_body, bpw=bpw, dim=dim),
        out_type=jax.ShapeDtypeStruct((3 * batch * dim,), jnp.int32),
        mesh=mesh,
        compiler_params=sc_params,
        scratch_types=[
            pltpu.VMEM((bpw,), jnp.int32),
            pltpu.VMEM((3 * bpw,), jnp.int32),
            pltpu.VMEM((3 * bpw,), jnp.int32),
            pltpu.VMEM((3 * bpw,), jnp.int32),
            pltpu.VMEM((nel,), jnp.int32),
            pltpu.SemaphoreType.DMA,
        ],
    )(lab3, uids)

    out = pl.kernel(
        functools.partial(_gather_body, nel=nel),
        out_type=jax.ShapeDtypeStruct((3 * batch * dim,), jnp.float32),
        mesh=mesh,
        compiler_params=sc_params,
        scratch_types=[
            pltpu.VMEM((nel,), jnp.int32),
            pltpu.VMEM((nel,), jnp.float32),
            pltpu.SemaphoreType.DMA,
        ],
    )(embf, idx2)
    return out.reshape(batch, 3 * dim)


# speedup vs baseline: 11.5726x; 1.0013x over previous
import functools

import jax
import jax.numpy as jnp
from jax import lax
from jax.experimental import pallas as pl
from jax.experimental.pallas import tpu as pltpu
from jax.experimental.pallas import tpu_sc as plsc

_NUM_USERS = 100000
_NC, _NS = 2, 16
_NW = _NC * _NS
_L = 16
_WCH = 124928


def _idx_body(lab3_hbm, uids_hbm, idx2_hbm,
              uids_v, idx1_v, ids_v, idx2_v, sem, bpw, dim):
    wid = lax.axis_index("s") * _NC + lax.axis_index("c")
    base = wid * bpw
    n3 = 3 * bpw
    nel = n3 * dim

    pltpu.sync_copy(uids_hbm.at[pl.ds(base, bpw)], uids_v)

    for j in range(3):
        for k in range(bpw // _L):
            p = j * bpw + k * _L
            u = uids_v[pl.ds(k * _L, _L)]
            idx1_v[pl.ds(p, _L)] = u + j * _NUM_USERS

    pltpu.async_copy(lab3_hbm.at[idx1_v], ids_v, sem).wait()

    three = jnp.full((_L,), 3, jnp.int32)
    iota = lax.iota(jnp.int32, _L)
    wvec = jnp.full((_L,), _WCH, jnp.int32)
    dconst = (lax.shift_right_logical(iota, 3) * (8 * _WCH)
              + lax.bitwise_and(iota, 7) * 128)

    def expand(k, _):
        pos = iota + k * _L
        b = lax.div(pos, three)
        j = lax.rem(pos, three)
        e = plsc.load_gather(ids_v, [j * bpw + b])
        ch = lax.div(e, wvec)
        r = lax.rem(e, wvec)
        s = (ch * (16 * _WCH)
             + lax.shift_left(lax.shift_right_logical(r, 7), 10)
             + lax.bitwise_and(r, 127))
        for m in range(_L):
            sv = lax.gather(
                s, jnp.full((_L, 1), m, jnp.int32),
                lax.GatherDimensionNumbers(
                    offset_dims=(), collapsed_slice_dims=(0,),
                    start_index_map=(0,)),
                (1,), mode=lax.GatherScatterMode.PROMISE_IN_BOUNDS)
            idx2_v[pl.ds(k * _L * _L + m * _L, _L)] = sv + dconst
        return _

    lax.fori_loop(0, n3 // _L, expand, 0)

    pltpu.sync_copy(idx2_v, idx2_hbm.at[pl.ds(wid * nel, nel)])


def _gather_body(embf_hbm, idx2_hbm, out_hbm, idx2_v, emb_v, sem, nel):
    wid = lax.axis_index("s") * _NC + lax.axis_index("c")
    pltpu.sync_copy(idx2_hbm.at[pl.ds(wid * nel, nel)], idx2_v)
    pltpu.async_copy(embf_hbm.at[idx2_v], emb_v, sem).wait()
    pltpu.sync_copy(emb_v, out_hbm.at[pl.ds(wid * nel, nel)])


def _retile_body(in_ref, out_ref):
    out_ref[...] = (in_ref[...]
                    .reshape(8, _WCH // 128, 128)
                    .swapaxes(0, 1)
                    .reshape(8 * _WCH // 128, 128))


def _retile_tc(emb_t):
    dim, vocab = emb_t.shape
    nch = (vocab + _WCH - 1) // _WCH
    rows = 8 * _WCH // 128
    return pl.pallas_call(
        _retile_body,
        grid=(dim // 8, nch),
        in_specs=[pl.BlockSpec((8, _WCH), lambda g, c: (g, c))],
        out_specs=pl.BlockSpec((rows, 128), lambda g, c: (c * (dim // 8) + g, 0)),
        out_shape=jax.ShapeDtypeStruct(((dim // 8) * nch * rows, 128),
                                       jnp.float32),
    )(emb_t)


@jax.jit
def kernel(uids, train_labels, item_embeddings):
    batch = uids.shape[0]
    dim = item_embeddings.shape[1]
    hist = train_labels.shape[1]
    bpw = batch // _NW
    nel = 3 * bpw * dim
    sc_params = pltpu.CompilerParams(
        needs_layout_passes=False, use_tc_tiling_on_sc=False)
    mesh = plsc.VectorSubcoreMesh(core_axis_name="c", subcore_axis_name="s")

    lab3 = train_labels.T[hist - 3:hist].reshape(-1)
    embf = _retile_tc(item_embeddings.T).reshape(-1)

    idx2 = pl.kernel(
        functools.partial(_idx_body, bpw=bpw, dim=dim),
        out_type=jax.ShapeDtypeStruct((3 * batch * dim,), jnp.int32),
        mesh=mesh,
        compiler_params=sc_params,
        scratch_types=[
            pltpu.VMEM((bpw,), jnp.int32),
            pltpu.VMEM((3 * bpw,), jnp.int32),
            pltpu.VMEM((3 * bpw,), jnp.int32),
            pltpu.VMEM((nel,), jnp.int32),
            pltpu.SemaphoreType.DMA,
        ],
    )(lab3, uids)

    out = pl.kernel(
        functools.partial(_gather_body, nel=nel),
        out_type=jax.ShapeDtypeStruct((3 * batch * dim,), jnp.float32),
        mesh=mesh,
        compiler_params=sc_params,
        scratch_types=[
            pltpu.VMEM((nel,), jnp.int32),
            pltpu.VMEM((nel,), jnp.float32),
            pltpu.SemaphoreType.DMA,
        ],
    )(embf, idx2)
    return out.reshape(batch, 3 * dim)
